# baseline (speedup 1.0000x reference)
import functools

import jax
import jax.numpy as jnp
from jax import lax
from jax.experimental import pallas as pl
from jax.experimental.pallas import tpu as pltpu

_BN_EPS = 1e-5


def _conv_stats_kernel(xp_ref, w_ref, yt_ref, stats_ref, *, oh, ow):
    ohw = oh * ow
    cin = xp_ref.shape[-1]
    taps = []
    for ki in range(3):
        for kj in range(3):
            tap = xp_ref[0, ki:ki + oh, kj:kj + ow, :]
            taps.append(tap.reshape(ohw, cin))
    xi = jnp.concatenate(taps, axis=1)
    acc = jnp.dot(xi, w_ref[...], preferred_element_type=jnp.float32)
    stats_ref[0, 0:1, :] = jnp.sum(acc, axis=0, keepdims=True)
    stats_ref[0, 1:2, :] = jnp.sum(acc * acc, axis=0, keepdims=True)
    yt_ref[0] = acc.astype(jnp.bfloat16).T


def _bn_relu_t_kernel(yt_ref, scale_ref, shift_ref, o_ref):
    y = yt_ref[0].astype(jnp.float32)
    o_ref[0] = jnp.maximum(y * scale_ref[...] + shift_ref[...], 0.0)


@jax.jit
def _forward(x_nchw, conv_weight, gamma, beta):
    N, Cin, H, W = x_nchw.shape
    Cout = conv_weight.shape[0]
    OH, OW = H, W
    OHW = OH * OW

    xb = jnp.transpose(x_nchw.astype(jnp.bfloat16), (0, 2, 3, 1))
    xp = jnp.pad(xb, ((0, 0), (1, 1), (1, 1), (0, 0)))
    w = jnp.transpose(conv_weight.astype(jnp.bfloat16), (2, 3, 1, 0))
    w = w.reshape(9 * Cin, Cout)

    kernel1 = functools.partial(_conv_stats_kernel, oh=OH, ow=OW)
    flops = 2 * N * OHW * (9 * Cin) * Cout
    bytes_acc = 2 * (xp.size + w.size + N * Cout * OHW) + 4 * N * 2 * Cout
    yt, stats = pl.pallas_call(
        kernel1,
        out_shape=(
            jax.ShapeDtypeStruct((N, Cout, OHW), jnp.bfloat16),
            jax.ShapeDtypeStruct((N, 2, Cout), jnp.float32),
        ),
        grid=(N,),
        in_specs=[
            pl.BlockSpec((1, H + 2, W + 2, Cin), lambda n: (n, 0, 0, 0)),
            pl.BlockSpec((9 * Cin, Cout), lambda n: (0, 0)),
        ],
        out_specs=(
            pl.BlockSpec((1, Cout, OHW), lambda n: (n, 0, 0)),
            pl.BlockSpec((1, 2, Cout), lambda n: (n, 0, 0)),
        ),
        compiler_params=pltpu.CompilerParams(dimension_semantics=("parallel",)),
        cost_estimate=pl.CostEstimate(flops=flops, transcendentals=0,
                                      bytes_accessed=bytes_acc),
    )(xp, w)

    count = float(N * OHW)
    ssum = jnp.sum(stats[:, 0, :], axis=0)
    ssq = jnp.sum(stats[:, 1, :], axis=0)
    mean = ssum / count
    var = jnp.maximum(ssq / count - mean * mean, 0.0)
    scale = gamma.astype(jnp.float32) * lax.rsqrt(var + _BN_EPS)
    shift = beta.astype(jnp.float32) - mean * scale

    out_t = pl.pallas_call(
        _bn_relu_t_kernel,
        out_shape=jax.ShapeDtypeStruct((N, Cout, OHW), jnp.float32),
        grid=(N,),
        in_specs=[
            pl.BlockSpec((1, Cout, OHW), lambda n: (n, 0, 0)),
            pl.BlockSpec((Cout, 1), lambda n: (0, 0)),
            pl.BlockSpec((Cout, 1), lambda n: (0, 0)),
        ],
        out_specs=pl.BlockSpec((1, Cout, OHW), lambda n: (n, 0, 0)),
        compiler_params=pltpu.CompilerParams(dimension_semantics=("parallel",)),
    )(yt, scale.reshape(Cout, 1), shift.reshape(Cout, 1))

    return out_t.reshape(N, Cout, OH, OW)


def kernel(x_nchw, conv_weight, gamma, beta):
    return _forward(x_nchw, conv_weight, gamma, beta)
